# baseline (speedup 1.0000x reference)
import functools

import jax
import jax.numpy as jnp
from jax import lax
from jax.experimental import pallas as pl
from jax.experimental.pallas import tpu as pltpu
from jax.experimental.pallas import tpu_sc as plsc

EMB_DIM = 32
PACK = 128 // EMB_DIM
NUM_WORKERS = 32
CHUNK = 128
NSLOT = 4
LA = 2


def _build(batch, fields):
    n_blk = batch // CHUNK
    n_total = fields * n_blk
    assert n_total % NUM_WORKERS == 0
    n_chunk = n_total // NUM_WORKERS
    assert n_chunk % NSLOT == 0 and n_chunk >= 2 * NSLOT

    mesh = plsc.VectorSubcoreMesh(core_axis_name="c", subcore_axis_name="s")

    @functools.partial(
        pl.kernel,
        mesh=mesh,
        out_type=jax.ShapeDtypeStruct((fields, EMB_DIM, batch), jnp.float32),
        scratch_types=(
            [pltpu.VMEM((n_chunk, CHUNK), jnp.int32)]
            + [pltpu.VMEM((NSLOT, CHUNK), jnp.int32)]
            + [pltpu.VMEM((CHUNK, 128), jnp.float32) for _ in range(NSLOT)]
            + [pltpu.VMEM((EMB_DIM, CHUNK), jnp.float32) for _ in range(NSLOT)]
            + [pltpu.SemaphoreType.DMA((NSLOT,)),
               pltpu.SemaphoreType.DMA((NSLOT,))]
        ),
        compiler_params=pltpu.CompilerParams(
            use_tc_tiling_on_sc=True, needs_layout_passes=False),
    )
    def emb(table_hbm, idx_hbm, out_hbm, idx_v, didx_v, *bufs):
        rows = bufs[:NSLOT]
        outs = bufs[NSLOT:2 * NSLOT]
        gsem, osem = bufs[2 * NSLOT], bufs[2 * NSLOT + 1]
        wid = lax.axis_index("s") * 2 + lax.axis_index("c")
        c0 = wid * n_chunk
        pltpu.sync_copy(idx_hbm.at[pl.ds(c0, n_chunk)], idx_v)
        iota16 = lax.iota(jnp.int32, 16)

        def prep(j, slot):
            for kb in range(CHUNK // 16):
                iv = idx_v[j, pl.ds(kb * 16, 16)]
                didx_v[slot, pl.ds(kb * 16, 16)] = lax.shift_right_logical(
                    iv, PACK // 2)

        def gstart(slot):
            pltpu.make_async_copy(
                table_hbm.at[didx_v.at[slot]], rows[slot], gsem.at[slot]
            ).start()

        def gwait(slot):
            pltpu.make_async_copy(
                table_hbm.at[didx_v.at[slot]], rows[slot], gsem.at[slot]
            ).wait()

        def extract(j, slot):
            def kb_body(kb, carry):
                iv = idx_v[j, pl.ds(kb * 16, 16)]
                colbase = (iv & (PACK - 1)) * EMB_DIM
                rowv = kb * 16 + iota16
                for e in range(EMB_DIM):
                    g = plsc.load_gather(rows[slot], [rowv, colbase + e])
                    outs[slot][e, pl.ds(kb * 16, 16)] = g
                return carry

            lax.fori_loop(0, CHUNK // 16, kb_body, 0)

        def ostart(j, slot):
            c = c0 + j
            f = c // n_blk
            ct = c % n_blk
            pltpu.make_async_copy(
                outs[slot], out_hbm.at[f, :, pl.ds(ct * CHUNK, CHUNK)],
                osem.at[slot],
            ).start()

        def owait(slot):
            pltpu.make_async_copy(
                outs[slot], out_hbm.at[0, :, pl.ds(0, CHUNK)], osem.at[slot]
            ).wait()

        for j in range(LA):
            prep(j, j)
            gstart(j)
        for j in range(NSLOT):
            a = j + LA
            prep(a, a % NSLOT)
            gstart(a % NSLOT)
            gwait(j % NSLOT)
            extract(j, j % NSLOT)
            ostart(j, j % NSLOT)

        def group(gi, carry):
            j0 = gi * NSLOT
            for b in range(NSLOT):
                j = j0 + b
                prep(j + LA, (b + LA) % NSLOT)
                gstart((b + LA) % NSLOT)
                gwait(b)
                owait(b)
                extract(j, b)
                ostart(j, b)
            return carry

        lax.fori_loop(1, n_chunk // NSLOT - 1, group, 0)

        for j in range(n_chunk - NSLOT, n_chunk):
            a = j + LA
            if a < n_chunk:
                prep(a, a % NSLOT)
                gstart(a % NSLOT)
            gwait(j % NSLOT)
            owait(j % NSLOT)
            extract(j, j % NSLOT)
            ostart(j, j % NSLOT)
        for s in range(NSLOT):
            owait(s)

    return emb


def kernel(x, weight):
    batch, fields = x.shape
    dict_size = weight.shape[0]
    table = weight.reshape(dict_size // PACK, 128)
    idx = x.T.reshape(fields * (batch // CHUNK), CHUNK)
    out = _build(batch, fields)(table, idx)
    return out.transpose(2, 0, 1)
